# baseline (speedup 1.0000x reference)
import jax
import jax.numpy as jnp
from jax.experimental import pallas as pl
from jax.experimental.pallas import tpu as pltpu

_MAX_ITER = 10
_TAU = 1.0
_CHUNK = 256


def _sinkhorn_kernel(t_hbm, out_hbm, k0_ref, k1_ref, k2_ref, s_ref,
                     in_sem0, in_sem1, in_sem2, out_sem0, out_sem1, out_sem2,
                     chunk_sems):
    b = pl.program_id(0)
    nb = pl.num_programs(0)
    n = k0_ref.shape[0]
    n_chunks = n // _CHUNK

    def rows(r):
        return pl.ds(r * _CHUNK, _CHUNK)

    def step(cur, nxt, prv, in_cur, in_nxt, out_cur, out_nxt, out_prv):
        @pl.when(b == 0)
        def _():
            for r in range(n_chunks):
                pltpu.make_async_copy(t_hbm.at[b, rows(r)], cur.at[rows(r)],
                                      chunk_sems.at[r]).start()

        @pl.when(b + 1 < nb)
        def _():
            @pl.when(b >= 2)
            def _():
                pltpu.make_async_copy(nxt, out_hbm.at[b - 2], out_nxt).wait()

            pltpu.make_async_copy(t_hbm.at[b + 1], nxt, in_nxt).start()

        @pl.when(b > 0)
        def _():
            pltpu.make_async_copy(t_hbm.at[b], cur, in_cur).wait()

        acc0 = jnp.zeros((1, n), dtype=jnp.float32)

        def init_chunk(r, acc):
            @pl.when(b == 0)
            def _():
                pltpu.make_async_copy(t_hbm.at[b, rows(r)], cur.at[rows(r)],
                                      chunk_sems.at[r]).wait()

            e = jnp.exp(cur[rows(r), :] * (1.0 / _TAU))
            cur[rows(r), :] = e
            s = jnp.sum(e, axis=1, keepdims=True)
            s_ref[rows(r), :] = s
            return acc + jnp.sum(e * (1.0 / s), axis=0, keepdims=True)

        v = 1.0 / jax.lax.fori_loop(0, n_chunks, init_chunk, acc0)

        def fused_pass(_, v):
            def chunk(r, acc):
                kc = cur[rows(r), :]
                s = jnp.sum(kc * v, axis=1, keepdims=True)
                s_ref[rows(r), :] = s
                return acc + jnp.sum(kc * (1.0 / s), axis=0, keepdims=True)

            return 1.0 / jax.lax.fori_loop(0, n_chunks, chunk, acc0, unroll=True)

        v = jax.lax.fori_loop(0, _MAX_ITER - 1, fused_pass, v)

        def prod_chunk(r, _):
            u = 1.0 / s_ref[rows(r), :]
            cur[rows(r), :] = cur[rows(r), :] * u * v
            return 0

        jax.lax.fori_loop(0, n_chunks, prod_chunk, 0)

        pltpu.make_async_copy(cur, out_hbm.at[b], out_cur).start()

        @pl.when(b == nb - 1)
        def _():
            @pl.when(b >= 2)
            def _():
                pltpu.make_async_copy(nxt, out_hbm.at[b - 2], out_nxt).wait()

            @pl.when(b >= 1)
            def _():
                pltpu.make_async_copy(prv, out_hbm.at[b - 1], out_prv).wait()

            pltpu.make_async_copy(cur, out_hbm.at[b], out_cur).wait()

    rem3 = jax.lax.rem(b, 3)

    @pl.when(rem3 == 0)
    def _():
        step(k0_ref, k1_ref, k2_ref,
             in_sem0, in_sem1, out_sem0, out_sem1, out_sem2)

    @pl.when(rem3 == 1)
    def _():
        step(k1_ref, k2_ref, k0_ref,
             in_sem1, in_sem2, out_sem1, out_sem2, out_sem0)

    @pl.when(rem3 == 2)
    def _():
        step(k2_ref, k0_ref, k1_ref,
             in_sem2, in_sem0, out_sem2, out_sem0, out_sem1)


@jax.jit
def kernel(t):
    b, n, m = t.shape
    return pl.pallas_call(
        _sinkhorn_kernel,
        grid=(b,),
        in_specs=[pl.BlockSpec(memory_space=pltpu.MemorySpace.HBM)],
        out_specs=pl.BlockSpec(memory_space=pltpu.MemorySpace.HBM),
        out_shape=jax.ShapeDtypeStruct((b, n, m), jnp.float32),
        scratch_shapes=[
            pltpu.VMEM((n, m), jnp.float32),
            pltpu.VMEM((n, m), jnp.float32),
            pltpu.VMEM((n, m), jnp.float32),
            pltpu.VMEM((n, 1), jnp.float32),
            pltpu.SemaphoreType.DMA,
            pltpu.SemaphoreType.DMA,
            pltpu.SemaphoreType.DMA,
            pltpu.SemaphoreType.DMA,
            pltpu.SemaphoreType.DMA,
            pltpu.SemaphoreType.DMA,
            pltpu.SemaphoreType.DMA((n // _CHUNK,)),
        ],
    )(t)


# speedup vs baseline: 1.0392x; 1.0392x over previous
import jax
import jax.numpy as jnp
from jax.experimental import pallas as pl
from jax.experimental.pallas import tpu as pltpu

_MAX_ITER = 10
_TAU = 1.0
_CHUNK = 512


def _sinkhorn_kernel(t_hbm, out_hbm, k0_ref, k1_ref, k2_ref, s_ref,
                     in_sem0, in_sem1, in_sem2, out_sem0, out_sem1, out_sem2,
                     chunk_sems):
    b = pl.program_id(0)
    nb = pl.num_programs(0)
    n = k0_ref.shape[0]
    n_chunks = n // _CHUNK

    def rows(r):
        return pl.ds(r * _CHUNK, _CHUNK)

    def step(cur, nxt, prv, in_cur, in_nxt, out_cur, out_nxt, out_prv):
        @pl.when(b == 0)
        def _():
            for r in range(n_chunks):
                pltpu.make_async_copy(t_hbm.at[b, rows(r)], cur.at[rows(r)],
                                      chunk_sems.at[r]).start()

        def prefetch_next():
            @pl.when(b >= 2)
            def _():
                pltpu.make_async_copy(nxt, out_hbm.at[b - 2], out_nxt).wait()

            pltpu.make_async_copy(t_hbm.at[b + 1], nxt, in_nxt).start()

        @pl.when(jnp.logical_and(b >= 1, b + 1 < nb))
        def _():
            prefetch_next()

        @pl.when(b > 0)
        def _():
            pltpu.make_async_copy(t_hbm.at[b], cur, in_cur).wait()

        acc0 = jnp.zeros((1, n), dtype=jnp.float32)

        def init_chunk(r, acc):
            @pl.when(b == 0)
            def _():
                pltpu.make_async_copy(t_hbm.at[b, rows(r)], cur.at[rows(r)],
                                      chunk_sems.at[r]).wait()

            e = jnp.exp(cur[rows(r), :] * (1.0 / _TAU))
            cur[rows(r), :] = e
            s = jnp.sum(e, axis=1, keepdims=True)
            s_ref[rows(r), :] = s
            return acc + jnp.sum(e * (1.0 / s), axis=0, keepdims=True)

        v = 1.0 / jax.lax.fori_loop(0, n_chunks, init_chunk, acc0, unroll=True)

        @pl.when(jnp.logical_and(b == 0, b + 1 < nb))
        def _():
            prefetch_next()

        def fused_pass(_, v):
            def chunk(r, acc):
                kc = cur[rows(r), :]
                s = jnp.sum(kc * v, axis=1, keepdims=True)
                s_ref[rows(r), :] = s
                return acc + jnp.sum(kc * (1.0 / s), axis=0, keepdims=True)

            return 1.0 / jax.lax.fori_loop(0, n_chunks, chunk, acc0, unroll=True)

        v = jax.lax.fori_loop(0, _MAX_ITER - 1, fused_pass, v)

        def prod_chunk(r, _):
            u = 1.0 / s_ref[rows(r), :]
            cur[rows(r), :] = cur[rows(r), :] * u * v
            return 0

        jax.lax.fori_loop(0, n_chunks, prod_chunk, 0, unroll=True)

        pltpu.make_async_copy(cur, out_hbm.at[b], out_cur).start()

        @pl.when(b == nb - 1)
        def _():
            @pl.when(b >= 2)
            def _():
                pltpu.make_async_copy(nxt, out_hbm.at[b - 2], out_nxt).wait()

            @pl.when(b >= 1)
            def _():
                pltpu.make_async_copy(prv, out_hbm.at[b - 1], out_prv).wait()

            pltpu.make_async_copy(cur, out_hbm.at[b], out_cur).wait()

    rem3 = jax.lax.rem(b, 3)

    @pl.when(rem3 == 0)
    def _():
        step(k0_ref, k1_ref, k2_ref,
             in_sem0, in_sem1, out_sem0, out_sem1, out_sem2)

    @pl.when(rem3 == 1)
    def _():
        step(k1_ref, k2_ref, k0_ref,
             in_sem1, in_sem2, out_sem1, out_sem2, out_sem0)

    @pl.when(rem3 == 2)
    def _():
        step(k2_ref, k0_ref, k1_ref,
             in_sem2, in_sem0, out_sem2, out_sem0, out_sem1)


@jax.jit
def kernel(t):
    b, n, m = t.shape
    return pl.pallas_call(
        _sinkhorn_kernel,
        grid=(b,),
        in_specs=[pl.BlockSpec(memory_space=pltpu.MemorySpace.HBM)],
        out_specs=pl.BlockSpec(memory_space=pltpu.MemorySpace.HBM),
        out_shape=jax.ShapeDtypeStruct((b, n, m), jnp.float32),
        scratch_shapes=[
            pltpu.VMEM((n, m), jnp.float32),
            pltpu.VMEM((n, m), jnp.float32),
            pltpu.VMEM((n, m), jnp.float32),
            pltpu.VMEM((n, 1), jnp.float32),
            pltpu.SemaphoreType.DMA,
            pltpu.SemaphoreType.DMA,
            pltpu.SemaphoreType.DMA,
            pltpu.SemaphoreType.DMA,
            pltpu.SemaphoreType.DMA,
            pltpu.SemaphoreType.DMA,
            pltpu.SemaphoreType.DMA((n // _CHUNK,)),
        ],
    )(t)
